# baseline (speedup 1.0000x reference)
import functools

import jax
import jax.numpy as jnp
from jax import lax
from jax.experimental import pallas as pl
from jax.experimental.pallas import tpu as pltpu
from jax.experimental.pallas import tpu_sc as plsc

NC = 2
NS = 16
NW = NC * NS
G = 128

N_PAD = 10240
ROWS_PER_TILE = N_PAD // NS
ZROWS = 128
BLK = 1280


def _zero_shared(zer_v, acc_sh, sid, width):
  def zrow(i, _):
    for off in range(0, width, 16):
      zer_v[i, pl.ds(off, 16)] = jnp.zeros((16,), jnp.float32)
    return 0
  lax.fori_loop(0, ZROWS, zrow, 0)
  base = pl.multiple_of(sid * ROWS_PER_TILE, ROWS_PER_TILE)
  for j in range(ROWS_PER_TILE // ZROWS):
    pltpu.sync_copy(zer_v, acc_sh.at[pl.ds(base + j * ZROWS, ZROWS)])


def _sc_degree(e4):
  groups = e4.shape[2]
  mesh = plsc.VectorSubcoreMesh(core_axis_name="c", subcore_axis_name="s")

  @functools.partial(
      pl.kernel,
      mesh=mesh,
      out_type=jax.ShapeDtypeStruct((NC, N_PAD), jnp.float32),
      scratch_types=[
          pltpu.VMEM((groups, G), jnp.int32),
          pltpu.VMEM((G,), jnp.float32),
          pltpu.VMEM((ROWS_PER_TILE,), jnp.float32),
          pltpu.VMEM_SHARED((N_PAD,), jnp.float32),
          pltpu.SemaphoreType.DMA,
      ],
  )
  def k(e_hbm, out_hbm, dstv, ones_v, zer_v, acc_sh, sem):
    cid = lax.axis_index("c")
    sid = lax.axis_index("s")
    wid = cid * NS + sid

    for i in range(G // 16):
      ones_v[pl.ds(i * 16, 16)] = jnp.full((16,), 1.0, jnp.float32)
    for i in range(ROWS_PER_TILE // 16):
      zer_v[pl.ds(i * 16, 16)] = jnp.zeros((16,), jnp.float32)
    base = pl.multiple_of(sid * ROWS_PER_TILE, ROWS_PER_TILE)
    pltpu.sync_copy(zer_v, acc_sh.at[pl.ds(base, ROWS_PER_TILE)])
    plsc.subcore_barrier()

    pltpu.sync_copy(e_hbm.at[1, wid], dstv)

    def body(g, _):
      pltpu.async_copy(ones_v, acc_sh.at[dstv.at[g]], sem, add=True)
      return 0
    lax.fori_loop(0, groups, body, 0)

    def drain(g, _):
      pltpu.make_async_copy(ones_v, acc_sh.at[dstv.at[0]], sem).wait()
      return 0
    lax.fori_loop(0, groups, drain, 0)

    plsc.subcore_barrier()
    pltpu.sync_copy(acc_sh.at[pl.ds(base, ROWS_PER_TILE)],
                    out_hbm.at[cid, pl.ds(base, ROWS_PER_TILE)])

  return k(e4)


def _sc_scatter(e4, table):
  groups = e4.shape[2]
  F = table.shape[1]
  mesh = plsc.VectorSubcoreMesh(core_axis_name="c", subcore_axis_name="s")

  R = 8
  A = 4
  assert groups >= R

  @functools.partial(
      pl.kernel,
      mesh=mesh,
      compiler_params=pltpu.CompilerParams(use_tc_tiling_on_sc=False),
      out_type=jax.ShapeDtypeStruct((NC, N_PAD, F), jnp.float32),
      scratch_types=[
          pltpu.VMEM((groups, G), jnp.int32),
          pltpu.VMEM((groups, G), jnp.int32),
          pltpu.VMEM((R, G, F), jnp.float32),
          pltpu.VMEM((ZROWS, F), jnp.float32),
          pltpu.VMEM_SHARED((N_PAD, F), jnp.float32),
          pltpu.SemaphoreType.DMA((R,)),
          pltpu.SemaphoreType.DMA((R,)),
      ],
  )
  def k(e_hbm, tab_hbm, out_hbm,
        srcv, dstv, rows, zer_v, acc_sh, sem_g, sem_s):
    cid = lax.axis_index("c")
    sid = lax.axis_index("s")
    wid = cid * NS + sid

    _zero_shared(zer_v, acc_sh, sid, F)
    plsc.subcore_barrier()

    pltpu.sync_copy(e_hbm.at[0, wid], srcv)
    pltpu.sync_copy(e_hbm.at[1, wid], dstv)

    for a in range(A):
      pltpu.async_copy(tab_hbm.at[srcv.at[a]], rows.at[a], sem_g.at[a])

    def body(g, _):
      @pl.when(g + A < groups)
      def _pref():
        bp = lax.rem(g + A, R)
        @pl.when(g + A >= R)
        def _wait_s():
          pltpu.make_async_copy(
              rows.at[bp], acc_sh.at[dstv.at[0]], sem_s.at[bp]).wait()
        pltpu.async_copy(tab_hbm.at[srcv.at[g + A]], rows.at[bp],
                         sem_g.at[bp])

      b = lax.rem(g, R)
      pltpu.make_async_copy(tab_hbm.at[srcv.at[g]], rows.at[b],
                            sem_g.at[b]).wait()
      pltpu.async_copy(rows.at[b], acc_sh.at[dstv.at[g]], sem_s.at[b],
                       add=True)
      return 0

    lax.fori_loop(0, groups, body, 0)

    for i in range(R):
      b = (groups - R + i) % R
      pltpu.make_async_copy(rows.at[b], acc_sh.at[dstv.at[0]],
                            sem_s.at[b]).wait()

    plsc.subcore_barrier()
    base = pl.multiple_of(sid * ROWS_PER_TILE, ROWS_PER_TILE)
    pltpu.sync_copy(acc_sh.at[pl.ds(base, ROWS_PER_TILE)],
                    out_hbm.at[cid, pl.ds(base, ROWS_PER_TILE)])

  return k(e4, table)


def _tc_matmul1(x, W1):
  n, d = x.shape
  h = W1.shape[1]

  def body(x_ref, w_ref, o_ref):
    o_ref[...] = jnp.dot(x_ref[...], w_ref[...],
                         preferred_element_type=jnp.float32)

  return pl.pallas_call(
      body,
      grid=(pl.cdiv(n, BLK),),
      in_specs=[pl.BlockSpec((BLK, d), lambda i: (i, 0)),
                pl.BlockSpec((d, h), lambda i: (0, 0))],
      out_specs=pl.BlockSpec((BLK, h), lambda i: (i, 0)),
      out_shape=jax.ShapeDtypeStruct((n, h), jnp.float32),
  )(x, W1)


def _tc_scale1(lin1, degt):
  n, h = lin1.shape

  def body(lin_ref, deg_ref, y_ref, dinv_ref):
    deg = jnp.dot(deg_ref[...], jnp.ones((NC, 1), jnp.float32),
                  preferred_element_type=jnp.float32) + 1.0
    dinv = lax.rsqrt(deg)
    y_ref[...] = lin_ref[...] * dinv
    dinv_ref[...] = dinv

  return pl.pallas_call(
      body,
      grid=(pl.cdiv(n, BLK),),
      in_specs=[pl.BlockSpec((BLK, h), lambda i: (i, 0)),
                pl.BlockSpec((BLK, NC), lambda i: (i, 0))],
      out_specs=[pl.BlockSpec((BLK, h), lambda i: (i, 0)),
                 pl.BlockSpec((BLK, 1), lambda i: (i, 0))],
      out_shape=[jax.ShapeDtypeStruct((n, h), jnp.float32),
                 jax.ShapeDtypeStruct((n, 1), jnp.float32)],
  )(lin1, degt)


def _tc_lin2(accp, y1, dinv, b1, W2):
  n, h = y1.shape
  c = W2.shape[1]
  cp = 48

  def body(a_ref, y1_ref, dinv_ref, b_ref, w_ref, y_ref):
    a = a_ref[0] + a_ref[1] + y1_ref[...]
    hid = jnp.maximum(a * dinv_ref[...] + b_ref[...], 0.0)
    lin = jnp.dot(hid, w_ref[...], preferred_element_type=jnp.float32)
    y = lin * dinv_ref[...]
    y_ref[...] = jnp.concatenate(
        [y, jnp.zeros((y.shape[0], cp - c), jnp.float32)], axis=1)

  return pl.pallas_call(
      body,
      grid=(pl.cdiv(n, BLK),),
      in_specs=[pl.BlockSpec((NC, BLK, h), lambda i: (0, i, 0)),
                pl.BlockSpec((BLK, h), lambda i: (i, 0)),
                pl.BlockSpec((BLK, 1), lambda i: (i, 0)),
                pl.BlockSpec((1, h), lambda i: (0, 0)),
                pl.BlockSpec((h, c), lambda i: (0, 0))],
      out_specs=pl.BlockSpec((BLK, cp), lambda i: (i, 0)),
      out_shape=jax.ShapeDtypeStruct((n, cp), jnp.float32),
  )(accp, y1, dinv, b1, W2)


def _tc_out(accp, y2, dinv, b2):
  n, cp = y2.shape
  c = b2.shape[1]

  def body(a_ref, y2_ref, dinv_ref, b_ref, o_ref):
    a = a_ref[0] + a_ref[1] + y2_ref[...]
    o = a[:, :c] * dinv_ref[...] + b_ref[...]
    m = jnp.max(o, axis=1, keepdims=True)
    s = o - m
    lse = jnp.log(jnp.sum(jnp.exp(s), axis=1, keepdims=True))
    o_ref[...] = s - lse

  return pl.pallas_call(
      body,
      grid=(pl.cdiv(n, BLK),),
      in_specs=[pl.BlockSpec((NC, BLK, cp), lambda i: (0, i, 0)),
                pl.BlockSpec((BLK, cp), lambda i: (i, 0)),
                pl.BlockSpec((BLK, 1), lambda i: (i, 0)),
                pl.BlockSpec((1, c), lambda i: (0, 0))],
      out_specs=pl.BlockSpec((BLK, c), lambda i: (i, 0)),
      out_shape=jax.ShapeDtypeStruct((n, c), jnp.float32),
  )(accp, y2, dinv, b2)


def kernel(x, edge_index, W1, b1, W2, b2):
  n, d = x.shape
  e = edge_index.shape[1]

  chunk = NW * G
  ep = chunk * ((e + chunk - 1) // chunk)
  groups = ep // chunk
  padv = jnp.broadcast_to(
      jnp.array([[0], [n]], edge_index.dtype), (2, ep - e))
  e4 = jnp.concatenate([edge_index, padv], axis=1).reshape(2, NW, groups, G)

  lin1 = _tc_matmul1(x, W1)
  degp = _sc_degree(e4)
  y1, dinv = _tc_scale1(lin1, degp.T)
  acc1 = _sc_scatter(e4, y1)
  y2 = _tc_lin2(acc1, y1, dinv, b1.reshape(1, -1), W2)
  acc2 = _sc_scatter(e4, y2)
  return _tc_out(acc2, y2, dinv, b2.reshape(1, -1))


# speedup vs baseline: 1.0411x; 1.0411x over previous
import functools

import jax
import jax.numpy as jnp
from jax import lax
from jax.experimental import pallas as pl
from jax.experimental.pallas import tpu as pltpu
from jax.experimental.pallas import tpu_sc as plsc

NC = 2
NS = 16
NW = NC * NS
G = 128

N_PAD = 10240
ROWS_PER_TILE = N_PAD // NS
ZROWS = 128
BLK = 1280


def _zero_shared(zer_v, acc_sh, sid, width):
  def zrow(i, _):
    for off in range(0, width, 16):
      zer_v[i, pl.ds(off, 16)] = jnp.zeros((16,), jnp.float32)
    return 0
  lax.fori_loop(0, ZROWS, zrow, 0)
  base = pl.multiple_of(sid * ROWS_PER_TILE, ROWS_PER_TILE)
  for j in range(ROWS_PER_TILE // ZROWS):
    pltpu.sync_copy(zer_v, acc_sh.at[pl.ds(base + j * ZROWS, ZROWS)])


def _sc_degree(e4):
  groups = e4.shape[2]
  mesh = plsc.VectorSubcoreMesh(core_axis_name="c", subcore_axis_name="s")

  @functools.partial(
      pl.kernel,
      mesh=mesh,
      out_type=jax.ShapeDtypeStruct((NC, N_PAD), jnp.float32),
      scratch_types=[
          pltpu.VMEM((groups, G), jnp.int32),
          pltpu.VMEM((G,), jnp.float32),
          pltpu.VMEM((ROWS_PER_TILE,), jnp.float32),
          pltpu.VMEM_SHARED((N_PAD,), jnp.float32),
          pltpu.SemaphoreType.DMA,
      ],
  )
  def k(e_hbm, out_hbm, dstv, ones_v, zer_v, acc_sh, sem):
    cid = lax.axis_index("c")
    sid = lax.axis_index("s")
    wid = cid * NS + sid

    for i in range(G // 16):
      ones_v[pl.ds(i * 16, 16)] = jnp.full((16,), 1.0, jnp.float32)
    for i in range(ROWS_PER_TILE // 16):
      zer_v[pl.ds(i * 16, 16)] = jnp.zeros((16,), jnp.float32)
    base = pl.multiple_of(sid * ROWS_PER_TILE, ROWS_PER_TILE)
    pltpu.sync_copy(zer_v, acc_sh.at[pl.ds(base, ROWS_PER_TILE)])
    plsc.subcore_barrier()

    pltpu.sync_copy(e_hbm.at[1, wid], dstv)

    def body(g, _):
      pltpu.async_copy(ones_v, acc_sh.at[dstv.at[g]], sem, add=True)
      return 0
    lax.fori_loop(0, groups, body, 0)

    def drain(g, _):
      pltpu.make_async_copy(ones_v, acc_sh.at[dstv.at[0]], sem).wait()
      return 0
    lax.fori_loop(0, groups, drain, 0)

    plsc.subcore_barrier()
    pltpu.sync_copy(acc_sh.at[pl.ds(base, ROWS_PER_TILE)],
                    out_hbm.at[cid, pl.ds(base, ROWS_PER_TILE)])

  return k(e4)


def _sc_scatter(e4, table):
  groups = e4.shape[2]
  F = table.shape[1]
  mesh = plsc.VectorSubcoreMesh(core_axis_name="c", subcore_axis_name="s")

  R = 8
  A = 4
  assert groups >= R

  @functools.partial(
      pl.kernel,
      mesh=mesh,
      compiler_params=pltpu.CompilerParams(use_tc_tiling_on_sc=False),
      out_type=jax.ShapeDtypeStruct((NC, N_PAD, F), jnp.float32),
      scratch_types=[
          pltpu.VMEM((groups, G), jnp.int32),
          pltpu.VMEM((groups, G), jnp.int32),
          pltpu.VMEM((R, G, F), jnp.float32),
          pltpu.VMEM((ZROWS, F), jnp.float32),
          pltpu.VMEM_SHARED((N_PAD, F), jnp.float32),
          pltpu.SemaphoreType.DMA((R,)),
          pltpu.SemaphoreType.DMA((R,)),
      ],
  )
  def k(e_hbm, tab_hbm, out_hbm,
        srcv, dstv, rows, zer_v, acc_sh, sem_g, sem_s):
    cid = lax.axis_index("c")
    sid = lax.axis_index("s")
    wid = cid * NS + sid

    _zero_shared(zer_v, acc_sh, sid, F)
    plsc.subcore_barrier()

    pltpu.sync_copy(e_hbm.at[0, wid], srcv)
    pltpu.sync_copy(e_hbm.at[1, wid], dstv)

    for a in range(A):
      pltpu.async_copy(tab_hbm.at[srcv.at[a]], rows.at[a], sem_g.at[a])

    def body(g, _):
      @pl.when(g + A < groups)
      def _pref():
        bp = lax.rem(g + A, R)
        @pl.when(g + A >= R)
        def _wait_s():
          pltpu.make_async_copy(
              rows.at[bp], acc_sh.at[dstv.at[0]], sem_s.at[bp]).wait()
        pltpu.async_copy(tab_hbm.at[srcv.at[g + A]], rows.at[bp],
                         sem_g.at[bp])

      b = lax.rem(g, R)
      pltpu.make_async_copy(tab_hbm.at[srcv.at[g]], rows.at[b],
                            sem_g.at[b]).wait()
      pltpu.async_copy(rows.at[b], acc_sh.at[dstv.at[g]], sem_s.at[b],
                       add=True)
      return 0

    lax.fori_loop(0, groups, body, 0)

    for i in range(R):
      b = (groups - R + i) % R
      pltpu.make_async_copy(rows.at[b], acc_sh.at[dstv.at[0]],
                            sem_s.at[b]).wait()

    plsc.subcore_barrier()
    base = pl.multiple_of(sid * ROWS_PER_TILE, ROWS_PER_TILE)
    pltpu.sync_copy(acc_sh.at[pl.ds(base, ROWS_PER_TILE)],
                    out_hbm.at[cid, pl.ds(base, ROWS_PER_TILE)])

  return k(e4, table)


def _tc_matmul1(x, W1):
  n, d = x.shape
  h = W1.shape[1]

  def body(x_ref, w_ref, o_ref):
    o_ref[...] = jnp.dot(x_ref[...], w_ref[...],
                         preferred_element_type=jnp.float32)

  return pl.pallas_call(
      body,
      grid=(pl.cdiv(n, BLK),),
      in_specs=[pl.BlockSpec((BLK, d), lambda i: (i, 0)),
                pl.BlockSpec((d, h), lambda i: (0, 0))],
      out_specs=pl.BlockSpec((BLK, h), lambda i: (i, 0)),
      out_shape=jax.ShapeDtypeStruct((n, h), jnp.float32),
  )(x, W1)


def _dinv_col(deg_ref):
  deg = lax.dot_general(deg_ref[...], jnp.ones((NC, 1), jnp.float32),
                        (((0,), (0,)), ((), ())),
                        preferred_element_type=jnp.float32) + 1.0
  return lax.rsqrt(deg)


def _tc_scale1(lin1, degp):
  n, h = lin1.shape

  def body(lin_ref, deg_ref, y_ref):
    y_ref[...] = lin_ref[...] * _dinv_col(deg_ref)

  return pl.pallas_call(
      body,
      grid=(pl.cdiv(n, BLK),),
      in_specs=[pl.BlockSpec((BLK, h), lambda i: (i, 0)),
                pl.BlockSpec((NC, BLK), lambda i: (0, i))],
      out_specs=pl.BlockSpec((BLK, h), lambda i: (i, 0)),
      out_shape=jax.ShapeDtypeStruct((n, h), jnp.float32),
  )(lin1, degp)


def _tc_lin2(accp, y1, degp, b1, W2):
  n, h = y1.shape
  c = W2.shape[1]
  cp = 48

  def body(a_ref, y1_ref, deg_ref, b_ref, w_ref, y_ref):
    dinv = _dinv_col(deg_ref)
    a = a_ref[0] + a_ref[1] + y1_ref[...]
    hid = jnp.maximum(a * dinv + b_ref[...], 0.0)
    lin = jnp.dot(hid, w_ref[...], preferred_element_type=jnp.float32)
    y = lin * dinv
    y_ref[...] = jnp.concatenate(
        [y, jnp.zeros((y.shape[0], cp - c), jnp.float32)], axis=1)

  return pl.pallas_call(
      body,
      grid=(pl.cdiv(n, BLK),),
      in_specs=[pl.BlockSpec((NC, BLK, h), lambda i: (0, i, 0)),
                pl.BlockSpec((BLK, h), lambda i: (i, 0)),
                pl.BlockSpec((NC, BLK), lambda i: (0, i)),
                pl.BlockSpec((1, h), lambda i: (0, 0)),
                pl.BlockSpec((h, c), lambda i: (0, 0))],
      out_specs=pl.BlockSpec((BLK, cp), lambda i: (i, 0)),
      out_shape=jax.ShapeDtypeStruct((n, cp), jnp.float32),
  )(accp, y1, degp, b1, W2)


def _tc_out(accp, y2, degp, b2):
  n, cp = y2.shape
  c = b2.shape[1]

  def body(a_ref, y2_ref, deg_ref, b_ref, o_ref):
    dinv = _dinv_col(deg_ref)
    a = a_ref[0] + a_ref[1] + y2_ref[...]
    o = a[:, :c] * dinv + b_ref[...]
    m = jnp.max(o, axis=1, keepdims=True)
    s = o - m
    lse = jnp.log(jnp.sum(jnp.exp(s), axis=1, keepdims=True))
    o_ref[...] = s - lse

  return pl.pallas_call(
      body,
      grid=(pl.cdiv(n, BLK),),
      in_specs=[pl.BlockSpec((NC, BLK, cp), lambda i: (0, i, 0)),
                pl.BlockSpec((BLK, cp), lambda i: (i, 0)),
                pl.BlockSpec((NC, BLK), lambda i: (0, i)),
                pl.BlockSpec((1, c), lambda i: (0, 0))],
      out_specs=pl.BlockSpec((BLK, c), lambda i: (i, 0)),
      out_shape=jax.ShapeDtypeStruct((n, c), jnp.float32),
  )(accp, y2, degp, b2)


def kernel(x, edge_index, W1, b1, W2, b2):
  n, d = x.shape
  e = edge_index.shape[1]

  chunk = NW * G
  ep = chunk * ((e + chunk - 1) // chunk)
  groups = ep // chunk
  pad_src = jnp.zeros((1, ep - e), edge_index.dtype)
  pad_dst = n + jax.lax.rem(
      jax.lax.iota(edge_index.dtype, ep - e), jnp.int32(N_PAD - n))[None]
  e4 = jnp.concatenate(
      [edge_index, jnp.concatenate([pad_src, pad_dst], axis=0)],
      axis=1).reshape(2, NW, groups, G)

  lin1 = _tc_matmul1(x, W1)
  degp = _sc_degree(e4)
  y1 = _tc_scale1(lin1, degp)
  acc1 = _sc_scatter(e4, y1)
  y2 = _tc_lin2(acc1, y1, degp, b1.reshape(1, -1), W2)
  acc2 = _sc_scatter(e4, y2)
  return _tc_out(acc2, y2, degp, b2.reshape(1, -1))
